# baseline (speedup 1.0000x reference)
import functools

import jax
import jax.numpy as jnp
from jax import lax
from jax.experimental import pallas as pl
from jax.experimental.pallas import tpu as pltpu
from jax.experimental.pallas import tpu_sc as plsc

E = 64
D = 768
DFF = 2048
N = 4096
T = 128
NT = N // T
MAXV = NT + E
MW = 128

_RB = 512
_NB = N // _RB
_SC_INFO = plsc.get_sparse_core_info()
_NC, _NS = _SC_INFO.num_cores, _SC_INFO.num_subcores
_NW = _NC * _NS
_BPW = N // _NW
_CH = _BPW // 2


def _col(row):
    r = lax.broadcasted_iota(jnp.int32, (E, E), 0)
    c = lax.broadcasted_iota(jnp.int32, (E, E), 1)
    return jnp.sum(jnp.where(r == c, row, 0.0), axis=1, keepdims=True)


def _router_meta_body(x_ref, gw_ref, eid_ref, meta_ref, counts_ref):
    s = pl.program_id(0)

    @pl.when(s == 0)
    def _():
        counts_ref[...] = jnp.zeros_like(counts_ref)

    @pl.when(s < _NB)
    def _():
        logits = jnp.dot(x_ref[...], gw_ref[...],
                         preferred_element_type=jnp.float32)
        eid_blk = jnp.argmax(logits, axis=-1).astype(jnp.int32)
        eid_ref[0, 0, :] = eid_blk
        oh = (eid_blk[:, None] == lax.broadcasted_iota(
            jnp.int32, (_RB, E), 1)).astype(jnp.float32)
        counts_ref[...] += jnp.sum(oh, axis=0, keepdims=True)

    @pl.when(s == _NB)
    def _():
        r = lax.broadcasted_iota(jnp.int32, (E, E), 0)
        c = lax.broadcasted_iota(jnp.int32, (E, E), 1)
        le = (r <= c).astype(jnp.float32)

        counts_row = counts_ref[...]
        counts_col = _col(counts_row)
        end_row = jnp.sum(counts_col * le, axis=0, keepdims=True)
        start_row = end_row - counts_row
        start_i = start_row.astype(jnp.int32)
        end_i = end_row.astype(jnp.int32)
        tlo = lax.shift_right_logical(start_i, 7)
        thi = lax.shift_right_logical(jnp.maximum(end_i - 1, 0), 7)
        nv = jnp.where(counts_row > 0, thi - tlo + 1, 0)

        nv_f = nv.astype(jnp.float32)
        nv_col = _col(nv_f)
        vcum_row = jnp.sum(nv_col * le, axis=0, keepdims=True)
        vcum_col = _col(vcum_row)
        vstart_row = vcum_row - nv_f
        nvis = vcum_row[:, E - 1:E]

        s_iota = lax.broadcasted_iota(jnp.int32, (E, MW), 1).astype(jnp.float32)
        mle = (vcum_col <= s_iota).astype(jnp.float32)
        ve = jnp.sum(mle, axis=0, keepdims=True)
        s_row = lax.broadcasted_iota(jnp.int32, (1, MW), 1).astype(jnp.float32)
        valid = s_row < nvis
        ve_c = jnp.minimum(ve, float(E - 1))
        e_last = jnp.sum((vcum_col <= nvis - 1.0).astype(jnp.float32),
                         axis=0, keepdims=True)

        e_iota = lax.broadcasted_iota(jnp.int32, (E, MW), 0).astype(jnp.float32)
        sel = (e_iota == ve_c).astype(jnp.float32)

        def gather(row):
            return jnp.sum(sel * _col(row), axis=0, keepdims=True)

        tlo_v = gather(tlo.astype(jnp.float32))
        vst_v = gather(vstart_row)
        lo_v = gather(start_row)
        hi_v = gather(end_row)

        vt = jnp.where(valid, tlo_v + s_row - vst_v, float(NT - 1))
        ve_out = jnp.where(valid, ve_c, e_last)
        vlo = jnp.where(valid, lo_v, 0.0)
        vhi = jnp.where(valid, hi_v, 0.0)
        vf = jnp.concatenate(
            [jnp.ones((1, 1), jnp.float32),
             (vt[:, 1:] != vt[:, :-1]).astype(jnp.float32)], axis=1)

        meta = jnp.concatenate(
            [vt, ve_out, vlo, vhi, vf, jnp.zeros((3, MW), jnp.float32)],
            axis=0)
        meta_ref[...] = meta.astype(jnp.int32)


def _router_meta(x, gate_w):
    eid, meta = pl.pallas_call(
        _router_meta_body,
        grid=(_NB + 1,),
        in_specs=[
            pl.BlockSpec((_RB, D), lambda s: (jnp.minimum(s, _NB - 1), 0)),
            pl.BlockSpec((D, E), lambda s: (0, 0)),
        ],
        out_specs=[
            pl.BlockSpec((1, 1, _RB), lambda s: (jnp.minimum(s, _NB - 1), 0, 0)),
            pl.BlockSpec((8, MW), lambda s: (0, 0)),
        ],
        out_shape=[
            jax.ShapeDtypeStruct((_NB, 1, _RB), jnp.int32),
            jax.ShapeDtypeStruct((8, MW), jnp.int32),
        ],
        scratch_shapes=[pltpu.VMEM((1, E), jnp.float32)],
        compiler_params=pltpu.CompilerParams(
            dimension_semantics=("arbitrary",)),
    )(x, gate_w)
    return eid.reshape(N), meta


def _make_sc_gather():
    mesh = plsc.VectorSubcoreMesh(core_axis_name="c", subcore_axis_name="s")

    @functools.partial(
        pl.kernel,
        mesh=mesh,
        out_type=jax.ShapeDtypeStruct((N, D), jnp.float32),
        scratch_types=[
            pltpu.VMEM((_CH,), jnp.int32),
            pltpu.VMEM((_CH,), jnp.int32),
            pltpu.VMEM((_CH, D), jnp.float32),
            pltpu.VMEM((_CH, D), jnp.float32),
            pltpu.SemaphoreType.DMA,
            pltpu.SemaphoreType.DMA,
        ],
    )
    def gather_k(table_hbm, idx_hbm, out_hbm, idx0, idx1, buf0, buf1, sem0, sem1):
        wid = lax.axis_index("s") * _NC + lax.axis_index("c")
        base = wid * _BPW
        pltpu.sync_copy(idx_hbm.at[pl.ds(base, _CH)], idx0)
        cp0 = pltpu.async_copy(table_hbm.at[idx0], buf0, sem0)
        pltpu.sync_copy(idx_hbm.at[pl.ds(base + _CH, _CH)], idx1)
        cp1 = pltpu.async_copy(table_hbm.at[idx1], buf1, sem1)
        cp0.wait()
        pltpu.sync_copy(buf0, out_hbm.at[pl.ds(base, _CH)])
        cp1.wait()
        pltpu.sync_copy(buf1, out_hbm.at[pl.ds(base + _CH, _CH)])

    return gather_k


_sc_gather = _make_sc_gather()


def _make_sc_scatter():
    mesh = plsc.VectorSubcoreMesh(core_axis_name="c", subcore_axis_name="s")

    @functools.partial(
        pl.kernel,
        mesh=mesh,
        out_type=jax.ShapeDtypeStruct((N, D), jnp.float32),
        scratch_types=[
            pltpu.VMEM((_CH,), jnp.int32),
            pltpu.VMEM((_CH,), jnp.int32),
            pltpu.VMEM((_CH, D), jnp.float32),
            pltpu.VMEM((_CH, D), jnp.float32),
            pltpu.SemaphoreType.DMA,
            pltpu.SemaphoreType.DMA,
            pltpu.SemaphoreType.DMA,
            pltpu.SemaphoreType.DMA,
        ],
    )
    def scatter_k(table_hbm, idx_hbm, out_hbm, idx0, idx1, buf0, buf1,
                  sl0, sl1, ss0, ss1):
        wid = lax.axis_index("s") * _NC + lax.axis_index("c")
        base = wid * _BPW
        pltpu.sync_copy(idx_hbm.at[pl.ds(base, _CH)], idx0)
        pltpu.sync_copy(idx_hbm.at[pl.ds(base + _CH, _CH)], idx1)
        cl0 = pltpu.async_copy(table_hbm.at[pl.ds(base, _CH)], buf0, sl0)
        cl1 = pltpu.async_copy(table_hbm.at[pl.ds(base + _CH, _CH)], buf1, sl1)
        cl0.wait()
        cs0 = pltpu.async_copy(buf0, out_hbm.at[idx0], ss0)
        cl1.wait()
        cs1 = pltpu.async_copy(buf1, out_hbm.at[idx1], ss1)
        cs0.wait()
        cs1.wait()

    return scatter_k


_sc_scatter = _make_sc_scatter()


_FS = 2
_DC = D // _FS
_FC = DFF // _FS


def _ffn_body(meta_ref, x_ref, *rest):
    w1_refs = rest[:_FS]
    w3_refs = rest[_FS:2 * _FS]
    w2_refs = rest[2 * _FS:3 * _FS]
    out_ref = rest[3 * _FS]
    s = pl.program_id(0)
    lo = meta_ref[2, s]
    hi = meta_ref[3, s]
    gid = meta_ref[0, s] * T + lax.broadcasted_iota(jnp.int32, (T, 1), 0)
    rowmask = ((gid >= lo) & (gid < hi)).astype(jnp.float32)
    x = x_ref[...] * rowmask
    a = sum(jnp.dot(x[:, f * _DC:(f + 1) * _DC], w1_refs[f][0],
                    preferred_element_type=jnp.float32) for f in range(_FS))
    bb = sum(jnp.dot(x[:, f * _DC:(f + 1) * _DC], w3_refs[f][0],
                     preferred_element_type=jnp.float32) for f in range(_FS))
    act = a * jax.nn.sigmoid(a) * bb
    contrib = sum(jnp.dot(act[:, f * _FC:(f + 1) * _FC], w2_refs[f][0],
                          preferred_element_type=jnp.float32)
                  for f in range(_FS))

    @pl.when(meta_ref[4, s] == 1)
    def _():
        out_ref[...] = contrib

    @pl.when(meta_ref[4, s] == 0)
    def _():
        out_ref[...] = out_ref[...] + contrib


def _grouped_ffn(x_sorted, w1, w2, w3, meta):
    w13_specs = [
        pl.BlockSpec((1, _DC, DFF),
                     (lambda f: lambda s, meta: (meta[1, s], f, 0))(f))
        for f in range(_FS)
    ]
    w2_specs = [
        pl.BlockSpec((1, _FC, D),
                     (lambda f: lambda s, meta: (meta[1, s], f, 0))(f))
        for f in range(_FS)
    ]
    grid_spec = pltpu.PrefetchScalarGridSpec(
        num_scalar_prefetch=1,
        grid=(MAXV,),
        in_specs=[pl.BlockSpec((T, D), lambda s, meta: (meta[0, s], 0))]
        + w13_specs + w13_specs + w2_specs,
        out_specs=pl.BlockSpec((T, D), lambda s, meta: (meta[0, s], 0)),
    )
    return pl.pallas_call(
        _ffn_body,
        grid_spec=grid_spec,
        out_shape=jax.ShapeDtypeStruct((N, D), jnp.float32),
        compiler_params=pltpu.CompilerParams(
            dimension_semantics=("arbitrary",)),
    )(meta, x_sorted, *([w1] * _FS), *([w3] * _FS), *([w2] * _FS))


def kernel(hidden_states, gate_w, w1, w2, w3):
    b, s, d = hidden_states.shape
    x = hidden_states.reshape(N, D)

    eid, meta = _router_meta(x, gate_w)

    order = jnp.argsort(eid).astype(jnp.int32)

    x_sorted = _sc_gather(x, order)
    y_sorted = _grouped_ffn(x_sorted, w1, w2, w3, meta)
    out = _sc_scatter(y_sorted, order)
    return out.reshape(b, s, d)


# speedup vs baseline: 1.0696x; 1.0696x over previous
import functools

import jax
import jax.numpy as jnp
from jax import lax
from jax.experimental import pallas as pl
from jax.experimental.pallas import tpu as pltpu
from jax.experimental.pallas import tpu_sc as plsc

E = 64
D = 768
DFF = 2048
N = 4096
T = 128
NT = N // T
MAXV = NT + E
MW = 128

_RB = 512
_NB = N // _RB
_SC_INFO = plsc.get_sparse_core_info()
_NC, _NS = _SC_INFO.num_cores, _SC_INFO.num_subcores
_NW = _NC * _NS
_BPW = N // _NW
_CH = _BPW // 2


def _col(row):
    r = lax.broadcasted_iota(jnp.int32, (E, E), 0)
    c = lax.broadcasted_iota(jnp.int32, (E, E), 1)
    return jnp.sum(jnp.where(r == c, row, 0.0), axis=1, keepdims=True)


def _router_meta_body(x_ref, gw_ref, eid_ref, meta_ref, counts_ref):
    s = pl.program_id(0)

    @pl.when(s == 0)
    def _():
        counts_ref[...] = jnp.zeros_like(counts_ref)

    @pl.when(s < _NB)
    def _():
        logits = jnp.dot(x_ref[...], gw_ref[...],
                         preferred_element_type=jnp.float32)
        eid_blk = jnp.argmax(logits, axis=-1).astype(jnp.int32)
        eid_ref[0, 0, :] = eid_blk
        oh = (eid_blk[:, None] == lax.broadcasted_iota(
            jnp.int32, (_RB, E), 1)).astype(jnp.float32)
        counts_ref[...] += jnp.sum(oh, axis=0, keepdims=True)

    @pl.when(s == _NB)
    def _():
        r = lax.broadcasted_iota(jnp.int32, (E, E), 0)
        c = lax.broadcasted_iota(jnp.int32, (E, E), 1)
        le = (r <= c).astype(jnp.float32)

        counts_row = counts_ref[...]
        counts_col = _col(counts_row)
        end_row = jnp.sum(counts_col * le, axis=0, keepdims=True)
        start_row = end_row - counts_row
        start_i = start_row.astype(jnp.int32)
        end_i = end_row.astype(jnp.int32)
        tlo = lax.shift_right_logical(start_i, 7)
        thi = lax.shift_right_logical(jnp.maximum(end_i - 1, 0), 7)
        nv = jnp.where(counts_row > 0, thi - tlo + 1, 0)

        nv_f = nv.astype(jnp.float32)
        nv_col = _col(nv_f)
        vcum_row = jnp.sum(nv_col * le, axis=0, keepdims=True)
        vcum_col = _col(vcum_row)
        vstart_row = vcum_row - nv_f
        nvis = vcum_row[:, E - 1:E]

        s_iota = lax.broadcasted_iota(jnp.int32, (E, MW), 1).astype(jnp.float32)
        mle = (vcum_col <= s_iota).astype(jnp.float32)
        ve = jnp.sum(mle, axis=0, keepdims=True)
        s_row = lax.broadcasted_iota(jnp.int32, (1, MW), 1).astype(jnp.float32)
        valid = s_row < nvis
        ve_c = jnp.minimum(ve, float(E - 1))
        e_last = jnp.sum((vcum_col <= nvis - 1.0).astype(jnp.float32),
                         axis=0, keepdims=True)

        e_iota = lax.broadcasted_iota(jnp.int32, (E, MW), 0).astype(jnp.float32)
        sel = (e_iota == ve_c).astype(jnp.float32)

        def gather(row):
            return jnp.sum(sel * _col(row), axis=0, keepdims=True)

        tlo_v = gather(tlo.astype(jnp.float32))
        vst_v = gather(vstart_row)
        lo_v = gather(start_row)
        hi_v = gather(end_row)

        vt = jnp.where(valid, tlo_v + s_row - vst_v, float(NT - 1))
        ve_out = jnp.where(valid, ve_c, e_last)
        vlo = jnp.where(valid, lo_v, 0.0)
        vhi = jnp.where(valid, hi_v, 0.0)
        vf = jnp.concatenate(
            [jnp.ones((1, 1), jnp.float32),
             (vt[:, 1:] != vt[:, :-1]).astype(jnp.float32)], axis=1)

        meta = jnp.concatenate(
            [vt, ve_out, vlo, vhi, vf, jnp.zeros((3, MW), jnp.float32)],
            axis=0)
        meta_ref[...] = meta.astype(jnp.int32)


def _router_meta(x, gate_w):
    eid, meta = pl.pallas_call(
        _router_meta_body,
        grid=(_NB + 1,),
        in_specs=[
            pl.BlockSpec((_RB, D), lambda s: (jnp.minimum(s, _NB - 1), 0)),
            pl.BlockSpec((D, E), lambda s: (0, 0)),
        ],
        out_specs=[
            pl.BlockSpec((1, 1, _RB), lambda s: (jnp.minimum(s, _NB - 1), 0, 0)),
            pl.BlockSpec((8, MW), lambda s: (0, 0)),
        ],
        out_shape=[
            jax.ShapeDtypeStruct((_NB, 1, _RB), jnp.int32),
            jax.ShapeDtypeStruct((8, MW), jnp.int32),
        ],
        scratch_shapes=[pltpu.VMEM((1, E), jnp.float32)],
        compiler_params=pltpu.CompilerParams(
            dimension_semantics=("arbitrary",)),
    )(x, gate_w)
    return eid.reshape(N), meta


def _make_sc_gather():
    mesh = plsc.VectorSubcoreMesh(core_axis_name="c", subcore_axis_name="s")

    @functools.partial(
        pl.kernel,
        mesh=mesh,
        out_type=jax.ShapeDtypeStruct((N, D), jnp.float32),
        scratch_types=[
            pltpu.VMEM((_CH,), jnp.int32),
            pltpu.VMEM((_CH,), jnp.int32),
            pltpu.VMEM((_CH, D), jnp.float32),
            pltpu.VMEM((_CH, D), jnp.float32),
            pltpu.SemaphoreType.DMA,
            pltpu.SemaphoreType.DMA,
        ],
    )
    def gather_k(table_hbm, idx_hbm, out_hbm, idx0, idx1, buf0, buf1, sem0, sem1):
        wid = lax.axis_index("s") * _NC + lax.axis_index("c")
        base = wid * _BPW
        pltpu.sync_copy(idx_hbm.at[pl.ds(base, _CH)], idx0)
        cp0 = pltpu.async_copy(table_hbm.at[idx0], buf0, sem0)
        pltpu.sync_copy(idx_hbm.at[pl.ds(base + _CH, _CH)], idx1)
        cp1 = pltpu.async_copy(table_hbm.at[idx1], buf1, sem1)
        cp0.wait()
        pltpu.sync_copy(buf0, out_hbm.at[pl.ds(base, _CH)])
        cp1.wait()
        pltpu.sync_copy(buf1, out_hbm.at[pl.ds(base + _CH, _CH)])

    return gather_k


_sc_gather = _make_sc_gather()


def _make_sc_scatter():
    mesh = plsc.VectorSubcoreMesh(core_axis_name="c", subcore_axis_name="s")

    @functools.partial(
        pl.kernel,
        mesh=mesh,
        out_type=jax.ShapeDtypeStruct((N, D), jnp.float32),
        scratch_types=[
            pltpu.VMEM((_CH,), jnp.int32),
            pltpu.VMEM((_CH,), jnp.int32),
            pltpu.VMEM((_CH, D), jnp.float32),
            pltpu.VMEM((_CH, D), jnp.float32),
            pltpu.SemaphoreType.DMA,
            pltpu.SemaphoreType.DMA,
            pltpu.SemaphoreType.DMA,
            pltpu.SemaphoreType.DMA,
        ],
    )
    def scatter_k(table_hbm, idx_hbm, out_hbm, idx0, idx1, buf0, buf1,
                  sl0, sl1, ss0, ss1):
        wid = lax.axis_index("s") * _NC + lax.axis_index("c")
        base = wid * _BPW
        pltpu.sync_copy(idx_hbm.at[pl.ds(base, _CH)], idx0)
        pltpu.sync_copy(idx_hbm.at[pl.ds(base + _CH, _CH)], idx1)
        cl0 = pltpu.async_copy(table_hbm.at[pl.ds(base, _CH)], buf0, sl0)
        cl1 = pltpu.async_copy(table_hbm.at[pl.ds(base + _CH, _CH)], buf1, sl1)
        cl0.wait()
        cs0 = pltpu.async_copy(buf0, out_hbm.at[idx0], ss0)
        cl1.wait()
        cs1 = pltpu.async_copy(buf1, out_hbm.at[idx1], ss1)
        cs0.wait()
        cs1.wait()

    return scatter_k


_sc_scatter = _make_sc_scatter()


def _ffn_body(meta_ref, x_ref, w1_ref, w3_ref, w2_ref, out_ref):
    s = pl.program_id(0)
    lo = meta_ref[2, s]
    hi = meta_ref[3, s]
    gid = meta_ref[0, s] * T + lax.broadcasted_iota(jnp.int32, (T, 1), 0)
    rowmask = ((gid >= lo) & (gid < hi)).astype(jnp.float32)
    x = x_ref[...] * rowmask
    a = jnp.dot(x, w1_ref[0], preferred_element_type=jnp.float32)
    b = jnp.dot(x, w3_ref[0], preferred_element_type=jnp.float32)
    act = a * jax.nn.sigmoid(a) * b
    contrib = jnp.dot(act, w2_ref[0], preferred_element_type=jnp.float32)

    @pl.when(meta_ref[4, s] == 1)
    def _():
        out_ref[...] = contrib

    @pl.when(meta_ref[4, s] == 0)
    def _():
        out_ref[...] = out_ref[...] + contrib


def _grouped_ffn(x_sorted, w1, w2, w3, meta):
    grid_spec = pltpu.PrefetchScalarGridSpec(
        num_scalar_prefetch=1,
        grid=(MAXV,),
        in_specs=[
            pl.BlockSpec((T, D), lambda s, meta: (meta[0, s], 0)),
            pl.BlockSpec((1, D, DFF), lambda s, meta: (meta[1, s], 0, 0)),
            pl.BlockSpec((1, D, DFF), lambda s, meta: (meta[1, s], 0, 0)),
            pl.BlockSpec((1, DFF, D), lambda s, meta: (meta[1, s], 0, 0)),
        ],
        out_specs=pl.BlockSpec((T, D), lambda s, meta: (meta[0, s], 0)),
    )
    return pl.pallas_call(
        _ffn_body,
        grid_spec=grid_spec,
        out_shape=jax.ShapeDtypeStruct((N, D), jnp.float32),
        compiler_params=pltpu.CompilerParams(
            dimension_semantics=("arbitrary",)),
    )(meta, x_sorted, w1, w3, w2)


def kernel(hidden_states, gate_w, w1, w2, w3):
    b, s, d = hidden_states.shape
    x = hidden_states.reshape(N, D)

    eid, meta = _router_meta(x, gate_w)

    order = jnp.argsort(eid).astype(jnp.int32)

    x_sorted = _sc_gather(x, order)
    y_sorted = _grouped_ffn(x_sorted, w1, w2, w3, meta)
    out = _sc_scatter(y_sorted, order)
    return out.reshape(b, s, d)


# speedup vs baseline: 1.0733x; 1.0035x over previous
import functools

import jax
import jax.numpy as jnp
from jax import lax
from jax.experimental import pallas as pl
from jax.experimental.pallas import tpu as pltpu
from jax.experimental.pallas import tpu_sc as plsc

E = 64
D = 768
DFF = 2048
N = 4096
T = 128
NT = N // T
MAXV = NT + E
MW = 128

_RB = 512
_NB = N // _RB
_SC_INFO = plsc.get_sparse_core_info()
_NC, _NS = _SC_INFO.num_cores, _SC_INFO.num_subcores
_NW = _NC * _NS
_BPW = N // _NW
_CH = _BPW // 2


def _col(row):
    r = lax.broadcasted_iota(jnp.int32, (E, E), 0)
    c = lax.broadcasted_iota(jnp.int32, (E, E), 1)
    return jnp.sum(jnp.where(r == c, row, 0.0), axis=1, keepdims=True)


def _router_meta_body(x_ref, gw_ref, pos_ref, meta_ref,
                      counts_ref, prefix_ref, eidv_ref, start_ref):
    s = pl.program_id(0)

    @pl.when(s == 0)
    def _():
        counts_ref[...] = jnp.zeros_like(counts_ref)

    @pl.when(s < _NB)
    def _():
        logits = jnp.dot(x_ref[...], gw_ref[...],
                         preferred_element_type=jnp.float32)
        eid_blk = jnp.argmax(logits, axis=-1).astype(jnp.int32)
        eidv_ref[pl.ds(s, 1), :] = eid_blk[None, :]
        prefix_ref[pl.ds(s, 1), :] = counts_ref[...]
        oh = (eid_blk[:, None] == lax.broadcasted_iota(
            jnp.int32, (_RB, E), 1)).astype(jnp.float32)
        counts_ref[...] += jnp.sum(oh, axis=0, keepdims=True)

    @pl.when(s > _NB)
    def _():
        b = s - _NB - 1
        eid_row = eidv_ref[pl.ds(b, 1), :].astype(jnp.float32)
        r5 = lax.broadcasted_iota(jnp.int32, (_RB, _RB), 0)
        c5 = lax.broadcasted_iota(jnp.int32, (_RB, _RB), 1)
        delta5 = (r5 == c5).astype(jnp.float32)
        lts5 = (r5 > c5).astype(jnp.float32)
        eid_col = jnp.sum(jnp.where(r5 == c5, eid_row, 0.0),
                          axis=1, keepdims=True)
        oh = (eid_col == lax.broadcasted_iota(
            jnp.int32, (_RB, E), 1).astype(jnp.float32)).astype(jnp.float32)
        cs = jnp.dot(lts5, oh, preferred_element_type=jnp.float32)
        base = start_ref[...] + prefix_ref[pl.ds(b, 1), :]
        pos_col = jnp.sum(oh * (cs + base), axis=1, keepdims=True)
        pos_row = jnp.sum(pos_col * delta5, axis=0, keepdims=True)
        pos_ref[0, 0, :] = pos_row[0].astype(jnp.int32)

    @pl.when(s == _NB)
    def _():
        r = lax.broadcasted_iota(jnp.int32, (E, E), 0)
        c = lax.broadcasted_iota(jnp.int32, (E, E), 1)
        le = (r <= c).astype(jnp.float32)

        counts_row = counts_ref[...]
        counts_col = _col(counts_row)
        end_row = jnp.sum(counts_col * le, axis=0, keepdims=True)
        start_row = end_row - counts_row
        start_ref[...] = start_row
        start_i = start_row.astype(jnp.int32)
        end_i = end_row.astype(jnp.int32)
        tlo = lax.shift_right_logical(start_i, 7)
        thi = lax.shift_right_logical(jnp.maximum(end_i - 1, 0), 7)
        nv = jnp.where(counts_row > 0, thi - tlo + 1, 0)

        nv_f = nv.astype(jnp.float32)
        nv_col = _col(nv_f)
        vcum_row = jnp.sum(nv_col * le, axis=0, keepdims=True)
        vcum_col = _col(vcum_row)
        vstart_row = vcum_row - nv_f
        nvis = vcum_row[:, E - 1:E]

        s_iota = lax.broadcasted_iota(jnp.int32, (E, MW), 1).astype(jnp.float32)
        mle = (vcum_col <= s_iota).astype(jnp.float32)
        ve = jnp.sum(mle, axis=0, keepdims=True)
        s_row = lax.broadcasted_iota(jnp.int32, (1, MW), 1).astype(jnp.float32)
        valid = s_row < nvis
        ve_c = jnp.minimum(ve, float(E - 1))
        e_last = jnp.sum((vcum_col <= nvis - 1.0).astype(jnp.float32),
                         axis=0, keepdims=True)

        e_iota = lax.broadcasted_iota(jnp.int32, (E, MW), 0).astype(jnp.float32)
        sel = (e_iota == ve_c).astype(jnp.float32)

        def gather(row):
            return jnp.sum(sel * _col(row), axis=0, keepdims=True)

        tlo_v = gather(tlo.astype(jnp.float32))
        vst_v = gather(vstart_row)
        lo_v = gather(start_row)
        hi_v = gather(end_row)

        vt = jnp.where(valid, tlo_v + s_row - vst_v, float(NT - 1))
        ve_out = jnp.where(valid, ve_c, e_last)
        vlo = jnp.where(valid, lo_v, 0.0)
        vhi = jnp.where(valid, hi_v, 0.0)
        vf = jnp.concatenate(
            [jnp.ones((1, 1), jnp.float32),
             (vt[:, 1:] != vt[:, :-1]).astype(jnp.float32)], axis=1)

        meta = jnp.concatenate(
            [vt, ve_out, vlo, vhi, vf, jnp.zeros((3, MW), jnp.float32)],
            axis=0)
        meta_ref[...] = meta.astype(jnp.int32)


def _router_meta(x, gate_w):
    pos, meta = pl.pallas_call(
        _router_meta_body,
        grid=(2 * _NB + 1,),
        in_specs=[
            pl.BlockSpec((_RB, D), lambda s: (jnp.minimum(s, _NB - 1), 0)),
            pl.BlockSpec((D, E), lambda s: (0, 0)),
        ],
        out_specs=[
            pl.BlockSpec((1, 1, _RB),
                         lambda s: (jnp.maximum(s - _NB - 1, 0), 0, 0)),
            pl.BlockSpec((8, MW), lambda s: (0, 0)),
        ],
        out_shape=[
            jax.ShapeDtypeStruct((_NB, 1, _RB), jnp.int32),
            jax.ShapeDtypeStruct((8, MW), jnp.int32),
        ],
        scratch_shapes=[
            pltpu.VMEM((1, E), jnp.float32),
            pltpu.VMEM((_NB, E), jnp.float32),
            pltpu.VMEM((_NB, _RB), jnp.int32),
            pltpu.VMEM((1, E), jnp.float32),
        ],
        compiler_params=pltpu.CompilerParams(
            dimension_semantics=("arbitrary",)),
    )(x, gate_w)
    return pos.reshape(N), meta


def _make_sc_gather():
    mesh = plsc.VectorSubcoreMesh(core_axis_name="c", subcore_axis_name="s")

    @functools.partial(
        pl.kernel,
        mesh=mesh,
        out_type=jax.ShapeDtypeStruct((N, D), jnp.float32),
        scratch_types=[
            pltpu.VMEM((_CH,), jnp.int32),
            pltpu.VMEM((_CH,), jnp.int32),
            pltpu.VMEM((_CH, D), jnp.float32),
            pltpu.VMEM((_CH, D), jnp.float32),
            pltpu.SemaphoreType.DMA,
            pltpu.SemaphoreType.DMA,
        ],
    )
    def gather_k(table_hbm, idx_hbm, out_hbm, idx0, idx1, buf0, buf1, sem0, sem1):
        wid = lax.axis_index("s") * _NC + lax.axis_index("c")
        base = wid * _BPW
        pltpu.sync_copy(idx_hbm.at[pl.ds(base, _CH)], idx0)
        cp0 = pltpu.async_copy(table_hbm.at[idx0], buf0, sem0)
        pltpu.sync_copy(idx_hbm.at[pl.ds(base + _CH, _CH)], idx1)
        cp1 = pltpu.async_copy(table_hbm.at[idx1], buf1, sem1)
        cp0.wait()
        pltpu.sync_copy(buf0, out_hbm.at[pl.ds(base, _CH)])
        cp1.wait()
        pltpu.sync_copy(buf1, out_hbm.at[pl.ds(base + _CH, _CH)])

    return gather_k


_sc_gather = _make_sc_gather()


def _make_sc_scatter():
    mesh = plsc.VectorSubcoreMesh(core_axis_name="c", subcore_axis_name="s")

    @functools.partial(
        pl.kernel,
        mesh=mesh,
        out_type=jax.ShapeDtypeStruct((N, D), jnp.float32),
        scratch_types=[
            pltpu.VMEM((_CH,), jnp.int32),
            pltpu.VMEM((_CH,), jnp.int32),
            pltpu.VMEM((_CH, D), jnp.float32),
            pltpu.VMEM((_CH, D), jnp.float32),
            pltpu.SemaphoreType.DMA,
            pltpu.SemaphoreType.DMA,
            pltpu.SemaphoreType.DMA,
            pltpu.SemaphoreType.DMA,
        ],
    )
    def scatter_k(table_hbm, idx_hbm, out_hbm, idx0, idx1, buf0, buf1,
                  sl0, sl1, ss0, ss1):
        wid = lax.axis_index("s") * _NC + lax.axis_index("c")
        base = wid * _BPW
        pltpu.sync_copy(idx_hbm.at[pl.ds(base, _CH)], idx0)
        pltpu.sync_copy(idx_hbm.at[pl.ds(base + _CH, _CH)], idx1)
        cl0 = pltpu.async_copy(table_hbm.at[pl.ds(base, _CH)], buf0, sl0)
        cl1 = pltpu.async_copy(table_hbm.at[pl.ds(base + _CH, _CH)], buf1, sl1)
        cl0.wait()
        cs0 = pltpu.async_copy(buf0, out_hbm.at[idx0], ss0)
        cl1.wait()
        cs1 = pltpu.async_copy(buf1, out_hbm.at[idx1], ss1)
        cs0.wait()
        cs1.wait()

    return scatter_k


_sc_scatter = _make_sc_scatter()


def _ffn_body(meta_ref, x_ref, w1_ref, w3_ref, w2_ref, out_ref):
    s = pl.program_id(0)
    lo = meta_ref[2, s]
    hi = meta_ref[3, s]
    gid = meta_ref[0, s] * T + lax.broadcasted_iota(jnp.int32, (T, 1), 0)
    rowmask = ((gid >= lo) & (gid < hi)).astype(jnp.float32)
    x = x_ref[...] * rowmask
    a = jnp.dot(x, w1_ref[0], preferred_element_type=jnp.float32)
    b = jnp.dot(x, w3_ref[0], preferred_element_type=jnp.float32)
    act = a * jax.nn.sigmoid(a) * b
    contrib = jnp.dot(act, w2_ref[0], preferred_element_type=jnp.float32)

    @pl.when(meta_ref[4, s] == 1)
    def _():
        out_ref[...] = contrib

    @pl.when(meta_ref[4, s] == 0)
    def _():
        out_ref[...] = out_ref[...] + contrib


def _grouped_ffn(x_sorted, w1, w2, w3, meta):
    grid_spec = pltpu.PrefetchScalarGridSpec(
        num_scalar_prefetch=1,
        grid=(MAXV,),
        in_specs=[
            pl.BlockSpec((T, D), lambda s, meta: (meta[0, s], 0)),
            pl.BlockSpec((1, D, DFF), lambda s, meta: (meta[1, s], 0, 0)),
            pl.BlockSpec((1, D, DFF), lambda s, meta: (meta[1, s], 0, 0)),
            pl.BlockSpec((1, DFF, D), lambda s, meta: (meta[1, s], 0, 0)),
        ],
        out_specs=pl.BlockSpec((T, D), lambda s, meta: (meta[0, s], 0)),
    )
    return pl.pallas_call(
        _ffn_body,
        grid_spec=grid_spec,
        out_shape=jax.ShapeDtypeStruct((N, D), jnp.float32),
        compiler_params=pltpu.CompilerParams(
            dimension_semantics=("arbitrary",)),
    )(meta, x_sorted, w1, w3, w2)


def kernel(hidden_states, gate_w, w1, w2, w3):
    b, s, d = hidden_states.shape
    x = hidden_states.reshape(N, D)

    pos, meta = _router_meta(x, gate_w)

    x_sorted = _sc_scatter(x, pos)
    y_sorted = _grouped_ffn(x_sorted, w1, w2, w3, meta)
    out = _sc_gather(y_sorted, pos)
    return out.reshape(b, s, d)
